# baseline (speedup 1.0000x reference)
import functools

import jax
import jax.numpy as jnp
from jax import lax
from jax.experimental import pallas as pl
from jax.experimental.pallas import tpu as pltpu
from jax.experimental.pallas import tpu_sc as plsc

CHUNK = 128
D = 64
TBLK = 4096
HSPLIT = 524288


def _tc_pair_table(table_t):
    d, vocab = table_t.shape
    n_blk = HSPLIT // TBLK
    max_blk = (vocab - 1) // TBLK

    def body(lo_ref, hi_ref, o_ref):
        o_ref[:, :d] = lo_ref[...].T
        o_ref[:, d:] = hi_ref[...].T

    return pl.pallas_call(
        body,
        grid=(n_blk,),
        in_specs=[
            pl.BlockSpec((d, TBLK), lambda i: (0, i)),
            pl.BlockSpec((d, TBLK), lambda i: (0, jnp.minimum(i + n_blk, max_blk))),
        ],
        out_specs=pl.BlockSpec((TBLK, 2 * D), lambda i: (i, 0)),
        out_shape=jax.ShapeDtypeStruct((HSPLIT, 2 * D), jnp.float32),
    )(table_t, table_t)


def _sc_gather_to_q(tokens2d, pair_table, n_t, n_bh):
    n_units, chunk = tokens2d.shape
    assert chunk == CHUNK and n_units == n_t * n_bh
    half = pair_table.shape[0]

    info = plsc.get_sparse_core_info()
    nc, ns = info.num_cores, info.num_subcores
    nw = nc * ns
    upw = n_units // nw
    assert upw * nw == n_units and upw % 2 == 0

    mesh = plsc.VectorSubcoreMesh(core_axis_name="c", subcore_axis_name="s")
    scratch = [
        pltpu.VMEM((upw, CHUNK), jnp.int32),
        pltpu.VMEM((upw, CHUNK), jnp.int32),
        pltpu.VMEM((CHUNK, 2 * D), jnp.float32),
        pltpu.VMEM((CHUNK, 2 * D), jnp.float32),
        pltpu.VMEM((8, 8, CHUNK), jnp.float32),
        pltpu.VMEM((8, 8, CHUNK), jnp.float32),
    ]
    scratch += [pltpu.SemaphoreType.DMA for _ in range(4)]

    @functools.partial(
        pl.kernel,
        mesh=mesh,
        out_type=jax.ShapeDtypeStruct((n_t * 8 * n_bh * 8, CHUNK), jnp.float32),
        scratch_types=scratch,
        compiler_params=pltpu.CompilerParams(needs_layout_passes=False),
    )
    def gather_kernel(tokens_hbm, table_hbm, out_hbm, tok_v, pair_v,
                      gbuf0, gbuf1, obuf0, obuf1,
                      gsem0, gsem1, wsem0, wsem1):
        gbufs = (gbuf0, gbuf1)
        obufs = (obuf0, obuf1)
        gsems = (gsem0, gsem1)
        wsems = (wsem0, wsem1)

        wid = lax.axis_index("s") * nc + lax.axis_index("c")
        unit0 = wid * upw

        pltpu.sync_copy(tokens_hbm.at[pl.ds(unit0, upw)], tok_v)

        def idx_body(r, carry):
            for g in range(CHUNK // 16):
                tok = tok_v[r, pl.ds(g * 16, 16)]
                pair_v[r, pl.ds(g * 16, 16)] = tok & (half - 1)
            return carry

        lax.fori_loop(0, upw, idx_body, 0)

        def g_copy(j, b):
            return pltpu.make_async_copy(
                table_hbm.at[pair_v.at[j]], gbufs[b], gsems[b])

        def w_copies(j, b):
            u = unit0 + j
            t = lax.shift_right_logical(u, 5)
            bh = u & (n_bh - 1)
            return [
                pltpu.make_async_copy(
                    obufs[b].at[dh],
                    out_hbm.at[pl.ds(((t * 8 + dh) * n_bh + bh) * 8, 8)],
                    wsems[b])
                for dh in range(8)
            ]

        def transpose_unit(j, b):
            gbuf = gbufs[b]
            obuf = obufs[b]

            def g_body(g, carry):
                g16 = g * 16
                tok = tok_v[j, pl.ds(g16, 16)]
                cb = lax.shift_left(lax.shift_right_logical(tok, 19), 6)
                rows = lax.iota(jnp.int32, 16) + g16
                for dh in range(8):
                    for dl in range(8):
                        k = dh * 8 + dl
                        vals = plsc.load_gather(gbuf, [rows, cb + k])
                        obuf[dh, dl, pl.ds(g16, 16)] = vals
                return carry

            lax.fori_loop(0, CHUNK // 16, g_body, 0)

        g_copy(0, 0).start()
        g_copy(1, 1).start()

        def unit_step(j, b, first, last):
            g_copy(j, b).wait()
            if not first:
                for c in w_copies(j - 2, b):
                    c.wait()
            transpose_unit(j, b)
            if not last:
                g_copy(j + 2, b).start()
            for c in w_copies(j, b):
                c.start()

        unit_step(0, 0, True, False)
        unit_step(1, 1, True, False)

        def body(k, carry):
            j = 2 + 2 * k
            unit_step(j, 0, False, False)
            unit_step(j + 1, 1, False, False)
            return carry

        lax.fori_loop(0, (upw - 4) // 2, body, 0)

        unit_step(upw - 2, 0, False, True)
        unit_step(upw - 1, 1, False, True)
        for c in w_copies(upw - 2, 0):
            c.wait()
        for c in w_copies(upw - 1, 1):
            c.wait()

    return gather_kernel(tokens2d, pair_table)


def kernel(tokens, embedding_table):
    batch, max_len = tokens.shape
    d = embedding_table.shape[1]
    n_bh = batch // CHUNK
    tok_units = tokens.T.reshape(max_len * n_bh, CHUNK)
    pair_table = _tc_pair_table(embedding_table.T)
    q = _sc_gather_to_q(tok_units, pair_table, max_len, n_bh)
    return (
        q.reshape(max_len, 8, n_bh, 8, CHUNK)
        .transpose(2, 4, 0, 1, 3)
        .reshape(batch, max_len, d)
    )


# speedup vs baseline: 2.0099x; 2.0099x over previous
import functools

import jax
import jax.numpy as jnp
from jax import lax
from jax.experimental import pallas as pl
from jax.experimental.pallas import tpu as pltpu
from jax.experimental.pallas import tpu_sc as plsc

CHUNK = 128
NBUF = 5


def _sc_gather(tokens2d, table):
    n_chunks, chunk = tokens2d.shape
    assert chunk == CHUNK
    d = table.shape[1]
    dout = 64

    info = plsc.get_sparse_core_info()
    nc, ns = info.num_cores, info.num_subcores
    nw = nc * ns
    chunks_per_w = n_chunks // nw
    assert chunks_per_w * nw == n_chunks
    assert chunks_per_w % NBUF == 0
    n_outer = chunks_per_w // NBUF
    total_rows = n_chunks * CHUNK

    mesh = plsc.VectorSubcoreMesh(core_axis_name="c", subcore_axis_name="s")
    scratch = [pltpu.VMEM((chunks_per_w, CHUNK), jnp.int32)]
    scratch += [pltpu.VMEM((CHUNK, d), jnp.float32) for _ in range(NBUF)]
    scratch += [pltpu.SemaphoreType.DMA for _ in range(2 * NBUF)]

    @functools.partial(
        pl.kernel,
        mesh=mesh,
        out_type=jax.ShapeDtypeStruct((total_rows, d), jnp.float32),
        scratch_types=scratch,
    )
    def gather_kernel(tokens_hbm, table_hbm, out_hbm, *refs):
        idx_v = refs[0]
        bufs = refs[1:1 + NBUF]
        gsems = refs[1 + NBUF:1 + 2 * NBUF]
        wsems = refs[1 + 2 * NBUF:1 + 3 * NBUF]

        wid = lax.axis_index("s") * nc + lax.axis_index("c")
        chunk0 = wid * chunks_per_w

        pltpu.sync_copy(tokens_hbm.at[pl.ds(chunk0, chunks_per_w)], idx_v)

        def g_copy(j, b):
            return pltpu.make_async_copy(
                table_hbm.at[idx_v.at[j]], bufs[b], gsems[b])

        def w_copy(j, b):
            return pltpu.make_async_copy(
                bufs[b],
                out_hbm.at[pl.ds((chunk0 + j) * CHUNK, CHUNK)],
                wsems[b])

        for b in range(NBUF):
            g_copy(b, b).start()

        def body(k, carry):
            j0 = k * NBUF
            for b in range(NBUF):
                g_copy(j0 + b, b).wait()
                w_copy(j0 + b, b).start()
            for b in range(NBUF):
                w_copy(j0 + b, b).wait()
                g_copy(j0 + NBUF + b, b).start()
            return carry

        lax.fori_loop(0, n_outer - 1, body, 0)

        j0 = (n_outer - 1) * NBUF
        for b in range(NBUF):
            g_copy(j0 + b, b).wait()
            w_copy(j0 + b, b).start()
        for b in range(NBUF):
            w_copy(j0 + b, b).wait()

    return gather_kernel(tokens2d, table)


TBLK = 8192


def _tc_transpose_pad(table_t):
    d, vocab = table_t.shape
    n_blk = (vocab + TBLK - 1) // TBLK

    def body(t_ref, o_ref):
        o_ref[:, :d] = t_ref[...].T

    return pl.pallas_call(
        body,
        grid=(n_blk,),
        in_specs=[pl.BlockSpec((d, TBLK), lambda i: (0, i))],
        out_specs=pl.BlockSpec((TBLK, CHUNK), lambda i: (i, 0)),
        out_shape=jax.ShapeDtypeStruct((vocab, CHUNK), jnp.float32),
    )(table_t)


def kernel(tokens, embedding_table):
    batch, max_len = tokens.shape
    d = embedding_table.shape[1]
    table128 = _tc_transpose_pad(embedding_table.T)
    flat = tokens.reshape(batch * max_len // CHUNK, CHUNK)
    out128 = _sc_gather(flat, table128)
    return out128.reshape(batch, max_len, CHUNK)[:, :, :d]


# speedup vs baseline: 2.0688x; 1.0293x over previous
import functools

import jax
import jax.numpy as jnp
from jax import lax
from jax.experimental import pallas as pl
from jax.experimental.pallas import tpu as pltpu
from jax.experimental.pallas import tpu_sc as plsc

CHUNK = 128
NBUF = 5


def _sc_gather(tokens2d, table):
    n_chunks, chunk = tokens2d.shape
    assert chunk == CHUNK
    d = table.shape[1]
    dout = 64

    info = plsc.get_sparse_core_info()
    nc, ns = info.num_cores, info.num_subcores
    nw = nc * ns
    chunks_per_w = n_chunks // nw
    assert chunks_per_w * nw == n_chunks
    assert chunks_per_w % NBUF == 0
    n_outer = chunks_per_w // NBUF
    total_rows = n_chunks * CHUNK

    mesh = plsc.VectorSubcoreMesh(core_axis_name="c", subcore_axis_name="s")
    scratch = [pltpu.VMEM((chunks_per_w, CHUNK), jnp.int32)]
    scratch += [pltpu.VMEM((CHUNK, d), jnp.float32) for _ in range(NBUF)]
    scratch += [pltpu.SemaphoreType.DMA for _ in range(2 * NBUF)]

    @functools.partial(
        pl.kernel,
        mesh=mesh,
        out_type=jax.ShapeDtypeStruct((total_rows, d), jnp.float32),
        scratch_types=scratch,
    )
    def gather_kernel(tokens_hbm, table_hbm, out_hbm, *refs):
        idx_v = refs[0]
        bufs = refs[1:1 + NBUF]
        gsems = refs[1 + NBUF:1 + 2 * NBUF]
        wsems = refs[1 + 2 * NBUF:1 + 3 * NBUF]

        wid = lax.axis_index("s") * nc + lax.axis_index("c")
        chunk0 = wid * chunks_per_w

        pltpu.sync_copy(tokens_hbm.at[pl.ds(chunk0, chunks_per_w)], idx_v)

        def g_copy(j, b):
            return pltpu.make_async_copy(
                table_hbm.at[idx_v.at[j]], bufs[b], gsems[b])

        def w_copy(j, b):
            return pltpu.make_async_copy(
                bufs[b],
                out_hbm.at[pl.ds((chunk0 + j) * CHUNK, CHUNK)],
                wsems[b])

        for b in range(NBUF):
            g_copy(b, b).start()

        def body(k, carry):
            j0 = k * NBUF
            for b in range(NBUF):
                g_copy(j0 + b, b).wait()
                w_copy(j0 + b, b).start()
            for b in range(NBUF):
                w_copy(j0 + b, b).wait()
                g_copy(j0 + NBUF + b, b).start()
            return carry

        lax.fori_loop(0, n_outer - 1, body, 0)

        j0 = (n_outer - 1) * NBUF
        for b in range(NBUF):
            g_copy(j0 + b, b).wait()
            w_copy(j0 + b, b).start()
        for b in range(NBUF):
            w_copy(j0 + b, b).wait()

    return gather_kernel(tokens2d, table)


TBLK = 16384


def _tc_transpose_pad(table_t):
    d, vocab = table_t.shape
    n_blk = (vocab + TBLK - 1) // TBLK

    def body(t_ref, o_ref):
        o_ref[:, :d] = t_ref[...].T

    return pl.pallas_call(
        body,
        grid=(n_blk,),
        in_specs=[pl.BlockSpec((d, TBLK), lambda i: (0, i))],
        out_specs=pl.BlockSpec((TBLK, CHUNK), lambda i: (i, 0)),
        out_shape=jax.ShapeDtypeStruct((vocab, CHUNK), jnp.float32),
    )(table_t)


def kernel(tokens, embedding_table):
    batch, max_len = tokens.shape
    d = embedding_table.shape[1]
    table128 = _tc_transpose_pad(embedding_table.T)
    flat = tokens.reshape(batch * max_len // CHUNK, CHUNK)
    out128 = _sc_gather(flat, table128)
    return out128.reshape(batch, max_len, CHUNK)[:, :, :d]


# speedup vs baseline: 2.0902x; 1.0103x over previous
import functools

import jax
import jax.numpy as jnp
from jax import lax
from jax.experimental import pallas as pl
from jax.experimental.pallas import tpu as pltpu
from jax.experimental.pallas import tpu_sc as plsc

CHUNK = 128
NBUF = 5


def _sc_gather(tokens2d, table):
    n_chunks, chunk = tokens2d.shape
    assert chunk == CHUNK
    d = table.shape[1]
    dout = 64

    info = plsc.get_sparse_core_info()
    nc, ns = info.num_cores, info.num_subcores
    nw = nc * ns
    chunks_per_w = n_chunks // nw
    assert chunks_per_w * nw == n_chunks
    assert chunks_per_w % NBUF == 0
    n_outer = chunks_per_w // NBUF
    total_rows = n_chunks * CHUNK

    mesh = plsc.VectorSubcoreMesh(core_axis_name="c", subcore_axis_name="s")
    scratch = [pltpu.VMEM((chunks_per_w, CHUNK), jnp.int32)]
    scratch += [pltpu.VMEM((CHUNK, d), jnp.float32) for _ in range(NBUF)]
    scratch += [pltpu.SemaphoreType.DMA for _ in range(2 * NBUF)]

    @functools.partial(
        pl.kernel,
        mesh=mesh,
        out_type=jax.ShapeDtypeStruct((total_rows, d), jnp.float32),
        scratch_types=scratch,
    )
    def gather_kernel(tokens_hbm, table_hbm, out_hbm, *refs):
        idx_v = refs[0]
        bufs = refs[1:1 + NBUF]
        gsems = refs[1 + NBUF:1 + 2 * NBUF]
        wsems = refs[1 + 2 * NBUF:1 + 3 * NBUF]

        wid = lax.axis_index("s") * nc + lax.axis_index("c")
        chunk0 = wid * chunks_per_w

        pltpu.sync_copy(tokens_hbm.at[pl.ds(chunk0, chunks_per_w)], idx_v)

        def g_copy(j, b):
            return pltpu.make_async_copy(
                table_hbm.at[idx_v.at[j]], bufs[b], gsems[b])

        def w_copy(j, b):
            return pltpu.make_async_copy(
                bufs[b],
                out_hbm.at[pl.ds((chunk0 + j) * CHUNK, CHUNK)],
                wsems[b])

        for b in range(NBUF):
            g_copy(b, b).start()

        def body(k, carry):
            j0 = k * NBUF
            for b in range(NBUF):
                g_copy(j0 + b, b).wait()
                w_copy(j0 + b, b).start()
            for b in range(NBUF):
                w_copy(j0 + b, b).wait()
                g_copy(j0 + NBUF + b, b).start()
            return carry

        lax.fori_loop(0, n_outer - 1, body, 0)

        j0 = (n_outer - 1) * NBUF
        for b in range(NBUF):
            g_copy(j0 + b, b).wait()
            w_copy(j0 + b, b).start()
        for b in range(NBUF):
            w_copy(j0 + b, b).wait()

    return gather_kernel(tokens2d, table)


TBLK = 32768


def _tc_transpose_pad(table_t):
    d, vocab = table_t.shape
    n_blk = (vocab + TBLK - 1) // TBLK

    def body(t_ref, o_ref):
        o_ref[:, :d] = t_ref[...].T

    return pl.pallas_call(
        body,
        grid=(n_blk,),
        in_specs=[pl.BlockSpec((d, TBLK), lambda i: (0, i))],
        out_specs=pl.BlockSpec((TBLK, CHUNK), lambda i: (i, 0)),
        out_shape=jax.ShapeDtypeStruct((vocab, CHUNK), jnp.float32),
    )(table_t)


def kernel(tokens, embedding_table):
    batch, max_len = tokens.shape
    d = embedding_table.shape[1]
    table128 = _tc_transpose_pad(embedding_table.T)
    flat = tokens.reshape(batch * max_len // CHUNK, CHUNK)
    out128 = _sc_gather(flat, table128)
    return out128.reshape(batch, max_len, CHUNK)[:, :, :d]
